# baseline (speedup 1.0000x reference)
import functools

import jax
import jax.numpy as jnp
from jax import lax
from jax.experimental import pallas as pl
from jax.experimental.pallas import tpu as pltpu
from jax.experimental.pallas import tpu_sc as plsc

N = 10000
E = 320000
NG = 64
NPAD = 10112
SLICE = NPAD // 16
E_PAD = 327680
EROWS = E_PAD // 128
ROWS_PER_TILE = EROWS // 32
CH = 2
CHB = 8
RMM = 1264
R = 2000
F32 = jnp.float32

_mesh = plsc.VectorSubcoreMesh(core_axis_name="c", subcore_axis_name="s")


def _deg_body(dst_hbm, out_hbm, dst_v, ones_v, zero_v, acc_sh):
    c = lax.axis_index("c")
    s = lax.axis_index("s")
    for j in range(640 // 16):
        zero_v[pl.ds(j * 16, 16)] = jnp.zeros((16,), F32)
    for j in range(128 // 16):
        ones_v[pl.ds(j * 16, 16)] = jnp.full((16,), 1.0, F32)
    pltpu.sync_copy(zero_v, acc_sh.at[pl.ds(s * 640, 640)])
    plsc.subcore_barrier()

    row0 = c * (EROWS // 2) + s * ROWS_PER_TILE

    def body(i, carry):
        pltpu.sync_copy(dst_hbm.at[pl.ds(row0 + i * CH, CH)], dst_v)
        for j in range(CH):
            pltpu.sync_copy(ones_v, acc_sh.at[dst_v.at[j]], add=True)
        return carry

    lax.fori_loop(0, ROWS_PER_TILE // CH, body, 0)
    plsc.subcore_barrier()
    pltpu.sync_copy(acc_sh.at[pl.ds(s * 640, 640)],
                    out_hbm.at[c, pl.ds(s * 640, 640)])


_deg_kernel = functools.partial(
    pl.kernel,
    out_type=jax.ShapeDtypeStruct((2, 10240), F32),
    mesh=_mesh,
    scratch_types=[
        pltpu.VMEM((CH, 128), jnp.int32),
        pltpu.VMEM((128,), F32),
        pltpu.VMEM((640,), F32),
        pltpu.VMEM_SHARED((10240,), F32),
    ],
)(_deg_body)


def _agg3_body(g_hbm, zeros_hbm, idx_hbm, out_hbm,
               idx_v, r0, r1, r2, acc_sh, g0, g1, g2, s0, s1, s2):
    c = lax.axis_index("c")
    s = lax.axis_index("s")
    @pl.when(c == 0)
    def _():
        pltpu.sync_copy(g_hbm.at[pl.ds(s * SLICE, SLICE)],
                        acc_sh.at[pl.ds(s * SLICE, SLICE)])

    @pl.when(c == 1)
    def _():
        pltpu.sync_copy(zeros_hbm.at[pl.ds(s * SLICE, SLICE)],
                        acc_sh.at[pl.ds(s * SLICE, SLICE)])

    plsc.subcore_barrier()

    rows = (r0, r1, r2)
    gsem = (g0, g1, g2)
    ssem = (s0, s1, s2)
    row_base = c * (EROWS // 2) + s * ROWS_PER_TILE

    def _wait_g(b):
        pltpu.make_async_copy(g_hbm.at[pl.ds(0, 128)], rows[b], gsem[b]).wait()

    def _wait_s(b):
        pltpu.make_async_copy(g_hbm.at[pl.ds(0, 128)], rows[b], ssem[b]).wait()

    def block(m, carry):
        pltpu.sync_copy(idx_hbm.at[pl.ds(row_base + 3 * m, 3)], idx_v)

        @pl.when(m > 0)
        def _():
            _wait_s(0)
            _wait_s(1)
        pltpu.async_copy(g_hbm.at[idx_v.at[0, 0]], rows[0], gsem[0])
        pltpu.async_copy(g_hbm.at[idx_v.at[1, 0]], rows[1], gsem[1])
        _wait_g(0)
        pltpu.async_copy(rows[0], acc_sh.at[idx_v.at[0, 1]], ssem[0], add=True)

        @pl.when(m > 0)
        def _():
            _wait_s(2)
        pltpu.async_copy(g_hbm.at[idx_v.at[2, 0]], rows[2], gsem[2])
        _wait_g(1)
        pltpu.async_copy(rows[1], acc_sh.at[idx_v.at[1, 1]], ssem[1], add=True)
        _wait_g(2)
        pltpu.async_copy(rows[2], acc_sh.at[idx_v.at[2, 1]], ssem[2], add=True)
        return carry

    lax.fori_loop(0, (ROWS_PER_TILE - 2) // 3, block, 0)
    pltpu.sync_copy(idx_hbm.at[pl.ds(row_base + ROWS_PER_TILE - 2, 2)],
                    idx_v.at[pl.ds(0, 2)])
    _wait_s(0)
    _wait_s(1)
    pltpu.async_copy(g_hbm.at[idx_v.at[0, 0]], rows[0], gsem[0])
    pltpu.async_copy(g_hbm.at[idx_v.at[1, 0]], rows[1], gsem[1])
    _wait_g(0)
    pltpu.async_copy(rows[0], acc_sh.at[idx_v.at[0, 1]], ssem[0], add=True)
    _wait_g(1)
    pltpu.async_copy(rows[1], acc_sh.at[idx_v.at[1, 1]], ssem[1], add=True)
    _wait_s(2)
    _wait_s(0)
    _wait_s(1)
    plsc.subcore_barrier()
    pltpu.sync_copy(acc_sh.at[pl.ds(s * SLICE, SLICE)],
                    out_hbm.at[c, pl.ds(s * SLICE, SLICE)])


_agg128 = functools.partial(
    pl.kernel,
    out_type=jax.ShapeDtypeStruct((2, NPAD, 128), F32),
    mesh=_mesh,
    compiler_params=pltpu.CompilerParams(use_tc_tiling_on_sc=False),
    scratch_types=(
        [pltpu.VMEM((3, 2, 128), jnp.int32)]
        + [pltpu.VMEM((128, 128), F32) for _ in range(3)]
        + [pltpu.VMEM_SHARED((NPAD, 128), F32)]
        + [pltpu.SemaphoreType.DMA] * 6
    ),
)(_agg3_body)


def _aggn_body(fw, g_hbm, zeros_hbm, idx_hbm, out_hbm,
               idx_v, r0, r1, r2, r3, acc_sh,
               g0, g1, g2, g3, s0, s1, s2, s3):
    c = lax.axis_index("c")
    s = lax.axis_index("s")
    @pl.when(c == 0)
    def _():
        pltpu.sync_copy(g_hbm.at[pl.ds(s * SLICE, SLICE)],
                        acc_sh.at[pl.ds(s * SLICE, SLICE)])

    @pl.when(c == 1)
    def _():
        pltpu.sync_copy(zeros_hbm.at[pl.ds(s * SLICE, SLICE)],
                        acc_sh.at[pl.ds(s * SLICE, SLICE)])

    plsc.subcore_barrier()

    rows = (r0, r1, r2, r3)
    gsem = (g0, g1, g2, g3)
    ssem = (s0, s1, s2, s3)
    row_base = c * (EROWS // 2) + s * ROWS_PER_TILE
    pltpu.sync_copy(idx_hbm.at[pl.ds(row_base, ROWS_PER_TILE)], idx_v)

    def _wait_g(b):
        pltpu.make_async_copy(g_hbm.at[pl.ds(0, 128)], rows[b], gsem[b]).wait()

    def _wait_s(b):
        pltpu.make_async_copy(g_hbm.at[pl.ds(0, 128)], rows[b], ssem[b]).wait()

    for j in range(3):
        pltpu.async_copy(g_hbm.at[idx_v.at[j, 0]], rows[j], gsem[j])

    def quad(k, carry):
        for u4 in range(4):
            j = 4 * k + u4
            b = u4
            b2 = (u4 + 3) % 4
            _wait_g(b)
            pltpu.async_copy(rows[b], acc_sh.at[idx_v.at[j, 1]],
                             ssem[b], add=True)
            j2 = j + 3

            @pl.when(jnp.logical_and(j >= 1, j2 < ROWS_PER_TILE))
            def _():
                _wait_s(b2)

            @pl.when(j2 < ROWS_PER_TILE)
            def _():
                pltpu.async_copy(g_hbm.at[idx_v.at[j2, 0]],
                                 rows[b2], gsem[b2])
        return carry

    lax.fori_loop(0, ROWS_PER_TILE // 4, quad, 0)
    for b in range(4):
        _wait_s(b)
    plsc.subcore_barrier()
    pltpu.sync_copy(acc_sh.at[pl.ds(s * SLICE, SLICE)],
                    out_hbm.at[c, pl.ds(s * SLICE, SLICE)])


def _make_aggn(fw):
    return functools.partial(
        pl.kernel,
        out_type=jax.ShapeDtypeStruct((2, NPAD, fw), F32),
        mesh=_mesh,
        compiler_params=pltpu.CompilerParams(use_tc_tiling_on_sc=False),
        scratch_types=(
            [pltpu.VMEM((ROWS_PER_TILE, 2, 128), jnp.int32)]
            + [pltpu.VMEM((128, fw), F32) for _ in range(4)]
            + [pltpu.VMEM_SHARED((NPAD, fw), F32)]
            + [pltpu.SemaphoreType.DMA] * 8
        ),
    )(functools.partial(_aggn_body, fw))


_agg64 = _make_aggn(64)
_agg32 = _make_aggn(32)


def _mm_first_body(x_ref, w_ref, deg_ref, o_ref):
    d = deg_ref[:, 0] + deg_ref[:, 1] + 1.0
    dinv = lax.rsqrt(d)
    g = jnp.dot(x_ref[...], w_ref[...], preferred_element_type=F32)
    o_ref[...] = g * dinv[:, None]


def _mm_mid_body(acc_ref, deg_ref, b_ref, w_ref, o_ref):
    d = deg_ref[:, 0] + deg_ref[:, 1] + 1.0
    dinv = lax.rsqrt(d)
    h = acc_ref[0] + acc_ref[1]
    a = jnp.maximum(dinv[:, None] * h + b_ref[...], 0.0)
    g = jnp.dot(a, w_ref[...], preferred_element_type=F32)
    o_ref[...] = g * dinv[:, None]


def _pool_body(acc_ref, deg_ref, b_ref, batch_ref, wfc_ref, bfc_ref,
               o_ref, pooled_scr):
    i = pl.program_id(0)

    @pl.when(i == 0)
    def _():
        pooled_scr[...] = jnp.zeros((NG, NG), F32)

    d = deg_ref[:, 0] + deg_ref[:, 1] + 1.0
    dinv = lax.rsqrt(d)
    h = acc_ref[0] + acc_ref[1]
    a = jnp.maximum(dinv[:, None] * h + b_ref[...], 0.0)
    ax = jnp.concatenate([a, jnp.ones((R, 32), F32)], axis=-1)
    gid = batch_ref[...]
    oh = (gid == lax.broadcasted_iota(jnp.int32, (R, NG), 1)).astype(F32)
    pooled_scr[...] += lax.dot_general(
        oh, ax, (((0,), (0,)), ((), ())), preferred_element_type=F32)

    @pl.when(i == pl.num_programs(0) - 1)
    def _():
        pooled = pooled_scr[...]
        counts = jnp.maximum(pooled[:, 32:33], 1.0)
        mean = pooled[:, :32] / counts
        o_ref[...] = jnp.dot(mean, wfc_ref[...],
                             preferred_element_type=F32) + bfc_ref[...]


def _mm_first(x, w, deg):
    return pl.pallas_call(
        _mm_first_body,
        grid=(NPAD // RMM,),
        in_specs=[
            pl.BlockSpec((RMM, x.shape[1]), lambda i: (i, 0)),
            pl.BlockSpec(w.shape, lambda i: (0, 0)),
            pl.BlockSpec((RMM, 2), lambda i: (i, 0)),
        ],
        out_specs=pl.BlockSpec((RMM, 128), lambda i: (i, 0)),
        out_shape=jax.ShapeDtypeStruct((NPAD, 128), F32),
    )(x, w, deg)


def _mm_mid(acc, deg, b, w):
    fin = acc.shape[2]
    fout = w.shape[1]
    return pl.pallas_call(
        _mm_mid_body,
        grid=(NPAD // RMM,),
        in_specs=[
            pl.BlockSpec((2, RMM, fin), lambda i: (0, i, 0)),
            pl.BlockSpec((RMM, 2), lambda i: (i, 0)),
            pl.BlockSpec(b.shape, lambda i: (0, 0)),
            pl.BlockSpec(w.shape, lambda i: (0, 0)),
        ],
        out_specs=pl.BlockSpec((RMM, fout), lambda i: (i, 0)),
        out_shape=jax.ShapeDtypeStruct((NPAD, fout), F32),
    )(acc, deg, b, w)


def _pool(acc, deg, b, batch2d, wfc, bfc):
    return pl.pallas_call(
        _pool_body,
        grid=(N // R,),
        in_specs=[
            pl.BlockSpec((2, R, 32), lambda i: (0, i, 0)),
            pl.BlockSpec((R, 2), lambda i: (i, 0)),
            pl.BlockSpec(b.shape, lambda i: (0, 0)),
            pl.BlockSpec((R, 1), lambda i: (i, 0)),
            pl.BlockSpec(wfc.shape, lambda i: (0, 0)),
            pl.BlockSpec(bfc.shape, lambda i: (0, 0)),
        ],
        out_specs=pl.BlockSpec((NG, 10), lambda i: (0, 0)),
        out_shape=jax.ShapeDtypeStruct((NG, 10), F32),
        scratch_shapes=[pltpu.VMEM((NG, NG), F32)],
    )(acc, deg, b, batch2d, wfc, bfc)


def kernel(x, edge_index, batch, W1, b1, W2, b2, W3, b3, W4, b4, W5, b5,
           Wfc, bfc):
    src = edge_index[0].astype(jnp.int32)
    dst = edge_index[1].astype(jnp.int32)
    npad = E_PAD - E
    pad_src = (jnp.arange(npad, dtype=jnp.int32) * 13) % N
    pad_dst = N + (jnp.arange(npad, dtype=jnp.int32) % (NPAD - N))
    src2 = jnp.concatenate([src, pad_src]).reshape(EROWS, 128)
    dst2 = jnp.concatenate([dst, pad_dst]).reshape(EROWS, 128)
    idx3 = jnp.stack([src2, dst2], axis=1)

    batch2d = batch.astype(jnp.int32).reshape(N, 1)
    zeros = jnp.zeros((NPAD, 128), F32)
    zeros64 = jnp.zeros((NPAD, 64), F32)
    zeros32 = jnp.zeros((NPAD, 32), F32)
    b1r = b1.reshape(1, -1)
    b2r = b2.reshape(1, -1)
    b3r = b3.reshape(1, -1)
    b4r = b4.reshape(1, -1)
    b5r = b5.reshape(1, -1)
    bfcr = bfc.reshape(1, -1)

    deg = _deg_kernel(dst2).T

    g = _mm_first(x, W1, deg)
    acc = _agg128(g, zeros, idx3)
    g = _mm_mid(acc, deg, b1r, W2)
    acc = _agg128(g, zeros, idx3)
    g = _mm_mid(acc, deg, b2r, W3)
    acc = _agg128(g, zeros, idx3)
    g = _mm_mid(acc, deg, b3r, W4)
    acc = _agg64(g, zeros64, idx3)
    g = _mm_mid(acc, deg, b4r, W5)
    acc = _agg32(g, zeros32, idx3)

    return _pool(acc, deg, b5r, batch2d, Wfc, bfcr)


# speedup vs baseline: 1.1386x; 1.1386x over previous
import functools

import jax
import jax.numpy as jnp
from jax import lax
from jax.experimental import pallas as pl
from jax.experimental.pallas import tpu as pltpu
from jax.experimental.pallas import tpu_sc as plsc

N = 10000
E = 320000
NG = 64
NPAD = 10112
SLICE = NPAD // 16
E_PAD = 327680
EROWS = E_PAD // 128
ROWS_PER_TILE = EROWS // 32
CH = 2
CHB = 8
RMM = 1264
R = 2000
F32 = jnp.float32

_mesh = plsc.VectorSubcoreMesh(core_axis_name="c", subcore_axis_name="s")


def _deg_body(dst_hbm, out_hbm, dst_v, ones_v, zero_v, acc_sh):
    c = lax.axis_index("c")
    s = lax.axis_index("s")
    for j in range(640 // 16):
        zero_v[pl.ds(j * 16, 16)] = jnp.zeros((16,), F32)
    for j in range(128 // 16):
        ones_v[pl.ds(j * 16, 16)] = jnp.full((16,), 1.0, F32)
    pltpu.sync_copy(zero_v, acc_sh.at[pl.ds(s * 640, 640)])
    plsc.subcore_barrier()

    row0 = c * (EROWS // 2) + s * ROWS_PER_TILE

    def body(i, carry):
        pltpu.sync_copy(dst_hbm.at[pl.ds(row0 + i * CH, CH)], dst_v)
        for j in range(CH):
            pltpu.sync_copy(ones_v, acc_sh.at[dst_v.at[j]], add=True)
        return carry

    lax.fori_loop(0, ROWS_PER_TILE // CH, body, 0)
    plsc.subcore_barrier()
    pltpu.sync_copy(acc_sh.at[pl.ds(s * 640, 640)],
                    out_hbm.at[c, pl.ds(s * 640, 640)])


_deg_kernel = functools.partial(
    pl.kernel,
    out_type=jax.ShapeDtypeStruct((2, 10240), F32),
    mesh=_mesh,
    scratch_types=[
        pltpu.VMEM((CH, 128), jnp.int32),
        pltpu.VMEM((128,), F32),
        pltpu.VMEM((640,), F32),
        pltpu.VMEM_SHARED((10240,), F32),
    ],
)(_deg_body)


def _agg2_body(g_hbm, zeros_hbm, idx_hbm, out_hbm,
               idx_v, r0, r1, acc_sh, g0, g1, s0, s1):
    c = lax.axis_index("c")
    s = lax.axis_index("s")
    @pl.when(c == 0)
    def _():
        pltpu.sync_copy(g_hbm.at[pl.ds(s * SLICE, SLICE)],
                        acc_sh.at[pl.ds(s * SLICE, SLICE)])

    @pl.when(c == 1)
    def _():
        pltpu.sync_copy(zeros_hbm.at[pl.ds(s * SLICE, SLICE)],
                        acc_sh.at[pl.ds(s * SLICE, SLICE)])

    plsc.subcore_barrier()

    rows = (r0, r1)
    gsem = (g0, g1)
    ssem = (s0, s1)
    row_base = c * (EROWS // 2) + s * ROWS_PER_TILE
    half = ROWS_PER_TILE // 2

    def _wait_g(b):
        pltpu.make_async_copy(g_hbm.at[pl.ds(0, 128)], rows[b], gsem[b]).wait()

    def _wait_s(b):
        pltpu.make_async_copy(g_hbm.at[pl.ds(0, 128)], rows[b], ssem[b]).wait()

    def _run_half(hbase, first_half):
        if not first_half:
            _wait_s(0)
            _wait_s(1)
        pltpu.sync_copy(idx_hbm.at[pl.ds(row_base + hbase, half)], idx_v)
        pltpu.async_copy(g_hbm.at[idx_v.at[0, 0]], rows[0], gsem[0])

        def body(k, carry):
            @pl.when(k > 0)
            def _():
                _wait_s(1)
            pltpu.async_copy(g_hbm.at[idx_v.at[2 * k + 1, 0]],
                             rows[1], gsem[1])
            _wait_g(0)
            pltpu.async_copy(rows[0], acc_sh.at[idx_v.at[2 * k, 1]],
                             ssem[0], add=True)

            @pl.when(k < half // 2 - 1)
            def _():
                _wait_s(0)
                pltpu.async_copy(g_hbm.at[idx_v.at[2 * k + 2, 0]],
                                 rows[0], gsem[0])
            _wait_g(1)
            pltpu.async_copy(rows[1], acc_sh.at[idx_v.at[2 * k + 1, 1]],
                             ssem[1], add=True)
            return carry

        lax.fori_loop(0, half // 2, body, 0)

    _run_half(0, True)
    _run_half(half, False)
    _wait_s(0)
    _wait_s(1)
    plsc.subcore_barrier()
    pltpu.sync_copy(acc_sh.at[pl.ds(s * SLICE, SLICE)],
                    out_hbm.at[c, pl.ds(s * SLICE, SLICE)])


_agg128 = functools.partial(
    pl.kernel,
    out_type=jax.ShapeDtypeStruct((2, NPAD, 128), F32),
    mesh=_mesh,
    compiler_params=pltpu.CompilerParams(use_tc_tiling_on_sc=False),
    scratch_types=(
        [pltpu.VMEM((ROWS_PER_TILE // 2, 2, 128), jnp.int32)]
        + [pltpu.VMEM((128, 128), F32) for _ in range(2)]
        + [pltpu.VMEM_SHARED((NPAD, 128), F32)]
        + [pltpu.SemaphoreType.DMA] * 4
    ),
)(_agg2_body)


def _aggn_body(fw, g_hbm, zeros_hbm, idx_hbm, out_hbm,
               idx_v, r0, r1, r2, r3, acc_sh,
               g0, g1, g2, g3, s0, s1, s2, s3):
    c = lax.axis_index("c")
    s = lax.axis_index("s")
    @pl.when(c == 0)
    def _():
        pltpu.sync_copy(g_hbm.at[pl.ds(s * SLICE, SLICE)],
                        acc_sh.at[pl.ds(s * SLICE, SLICE)])

    @pl.when(c == 1)
    def _():
        pltpu.sync_copy(zeros_hbm.at[pl.ds(s * SLICE, SLICE)],
                        acc_sh.at[pl.ds(s * SLICE, SLICE)])

    plsc.subcore_barrier()

    rows = (r0, r1, r2, r3)
    gsem = (g0, g1, g2, g3)
    ssem = (s0, s1, s2, s3)
    row_base = c * (EROWS // 2) + s * ROWS_PER_TILE
    pltpu.sync_copy(idx_hbm.at[pl.ds(row_base, ROWS_PER_TILE)], idx_v)

    def _wait_g(b):
        pltpu.make_async_copy(g_hbm.at[pl.ds(0, 128)], rows[b], gsem[b]).wait()

    def _wait_s(b):
        pltpu.make_async_copy(g_hbm.at[pl.ds(0, 128)], rows[b], ssem[b]).wait()

    for j in range(3):
        pltpu.async_copy(g_hbm.at[idx_v.at[j, 0]], rows[j], gsem[j])

    def quad(k, carry):
        for u4 in range(4):
            j = 4 * k + u4
            b = u4
            b2 = (u4 + 3) % 4
            _wait_g(b)
            pltpu.async_copy(rows[b], acc_sh.at[idx_v.at[j, 1]],
                             ssem[b], add=True)
            j2 = j + 3

            @pl.when(jnp.logical_and(j >= 1, j2 < ROWS_PER_TILE))
            def _():
                _wait_s(b2)

            @pl.when(j2 < ROWS_PER_TILE)
            def _():
                pltpu.async_copy(g_hbm.at[idx_v.at[j2, 0]],
                                 rows[b2], gsem[b2])
        return carry

    lax.fori_loop(0, ROWS_PER_TILE // 4, quad, 0)
    for b in range(4):
        _wait_s(b)
    plsc.subcore_barrier()
    pltpu.sync_copy(acc_sh.at[pl.ds(s * SLICE, SLICE)],
                    out_hbm.at[c, pl.ds(s * SLICE, SLICE)])


def _make_aggn(fw):
    return functools.partial(
        pl.kernel,
        out_type=jax.ShapeDtypeStruct((2, NPAD, fw), F32),
        mesh=_mesh,
        compiler_params=pltpu.CompilerParams(use_tc_tiling_on_sc=False),
        scratch_types=(
            [pltpu.VMEM((ROWS_PER_TILE, 2, 128), jnp.int32)]
            + [pltpu.VMEM((128, fw), F32) for _ in range(4)]
            + [pltpu.VMEM_SHARED((NPAD, fw), F32)]
            + [pltpu.SemaphoreType.DMA] * 8
        ),
    )(functools.partial(_aggn_body, fw))


_agg64 = _make_aggn(64)
_agg32 = _make_aggn(32)


def _mm_first_body(x_ref, w_ref, deg_ref, o_ref):
    d = deg_ref[:, 0] + deg_ref[:, 1] + 1.0
    dinv = lax.rsqrt(d)
    g = jnp.dot(x_ref[...], w_ref[...], preferred_element_type=F32)
    o_ref[...] = g * dinv[:, None]


def _mm_mid_body(acc_ref, deg_ref, b_ref, w_ref, o_ref):
    d = deg_ref[:, 0] + deg_ref[:, 1] + 1.0
    dinv = lax.rsqrt(d)
    h = acc_ref[0] + acc_ref[1]
    a = jnp.maximum(dinv[:, None] * h + b_ref[...], 0.0)
    g = jnp.dot(a, w_ref[...], preferred_element_type=F32)
    o_ref[...] = g * dinv[:, None]


def _pool_body(acc_ref, deg_ref, b_ref, batch_ref, wfc_ref, bfc_ref,
               o_ref, pooled_scr):
    i = pl.program_id(0)

    @pl.when(i == 0)
    def _():
        pooled_scr[...] = jnp.zeros((NG, NG), F32)

    d = deg_ref[:, 0] + deg_ref[:, 1] + 1.0
    dinv = lax.rsqrt(d)
    h = acc_ref[0] + acc_ref[1]
    a = jnp.maximum(dinv[:, None] * h + b_ref[...], 0.0)
    ax = jnp.concatenate([a, jnp.ones((R, 32), F32)], axis=-1)
    gid = batch_ref[...]
    oh = (gid == lax.broadcasted_iota(jnp.int32, (R, NG), 1)).astype(F32)
    pooled_scr[...] += lax.dot_general(
        oh, ax, (((0,), (0,)), ((), ())), preferred_element_type=F32)

    @pl.when(i == pl.num_programs(0) - 1)
    def _():
        pooled = pooled_scr[...]
        counts = jnp.maximum(pooled[:, 32:33], 1.0)
        mean = pooled[:, :32] / counts
        o_ref[...] = jnp.dot(mean, wfc_ref[...],
                             preferred_element_type=F32) + bfc_ref[...]


def _mm_first(x, w, deg):
    return pl.pallas_call(
        _mm_first_body,
        grid=(NPAD // RMM,),
        in_specs=[
            pl.BlockSpec((RMM, x.shape[1]), lambda i: (i, 0)),
            pl.BlockSpec(w.shape, lambda i: (0, 0)),
            pl.BlockSpec((RMM, 2), lambda i: (i, 0)),
        ],
        out_specs=pl.BlockSpec((RMM, 128), lambda i: (i, 0)),
        out_shape=jax.ShapeDtypeStruct((NPAD, 128), F32),
    )(x, w, deg)


def _mm_mid(acc, deg, b, w):
    fin = acc.shape[2]
    fout = w.shape[1]
    return pl.pallas_call(
        _mm_mid_body,
        grid=(NPAD // RMM,),
        in_specs=[
            pl.BlockSpec((2, RMM, fin), lambda i: (0, i, 0)),
            pl.BlockSpec((RMM, 2), lambda i: (i, 0)),
            pl.BlockSpec(b.shape, lambda i: (0, 0)),
            pl.BlockSpec(w.shape, lambda i: (0, 0)),
        ],
        out_specs=pl.BlockSpec((RMM, fout), lambda i: (i, 0)),
        out_shape=jax.ShapeDtypeStruct((NPAD, fout), F32),
    )(acc, deg, b, w)


def _pool(acc, deg, b, batch2d, wfc, bfc):
    return pl.pallas_call(
        _pool_body,
        grid=(N // R,),
        in_specs=[
            pl.BlockSpec((2, R, 32), lambda i: (0, i, 0)),
            pl.BlockSpec((R, 2), lambda i: (i, 0)),
            pl.BlockSpec(b.shape, lambda i: (0, 0)),
            pl.BlockSpec((R, 1), lambda i: (i, 0)),
            pl.BlockSpec(wfc.shape, lambda i: (0, 0)),
            pl.BlockSpec(bfc.shape, lambda i: (0, 0)),
        ],
        out_specs=pl.BlockSpec((NG, 10), lambda i: (0, 0)),
        out_shape=jax.ShapeDtypeStruct((NG, 10), F32),
        scratch_shapes=[pltpu.VMEM((NG, NG), F32)],
    )(acc, deg, b, batch2d, wfc, bfc)


def kernel(x, edge_index, batch, W1, b1, W2, b2, W3, b3, W4, b4, W5, b5,
           Wfc, bfc):
    src = edge_index[0].astype(jnp.int32)
    dst = edge_index[1].astype(jnp.int32)
    npad = E_PAD - E
    pad_src = (jnp.arange(npad, dtype=jnp.int32) * 13) % N
    pad_dst = N + (jnp.arange(npad, dtype=jnp.int32) % (NPAD - N))
    src2 = jnp.concatenate([src, pad_src]).reshape(EROWS, 128)
    dst2 = jnp.concatenate([dst, pad_dst]).reshape(EROWS, 128)
    idx3 = jnp.stack([src2, dst2], axis=1)

    batch2d = batch.astype(jnp.int32).reshape(N, 1)
    zeros = jnp.zeros((NPAD, 128), F32)
    zeros64 = jnp.zeros((NPAD, 64), F32)
    zeros32 = jnp.zeros((NPAD, 32), F32)
    b1r = b1.reshape(1, -1)
    b2r = b2.reshape(1, -1)
    b3r = b3.reshape(1, -1)
    b4r = b4.reshape(1, -1)
    b5r = b5.reshape(1, -1)
    bfcr = bfc.reshape(1, -1)

    deg = _deg_kernel(dst2).T

    g = _mm_first(x, W1, deg)
    acc = _agg128(g, zeros, idx3)
    g = _mm_mid(acc, deg, b1r, W2)
    acc = _agg128(g, zeros, idx3)
    g = _mm_mid(acc, deg, b2r, W3)
    acc = _agg128(g, zeros, idx3)
    g = _mm_mid(acc, deg, b3r, W4)
    acc = _agg64(g, zeros64, idx3)
    g = _mm_mid(acc, deg, b4r, W5)
    acc = _agg32(g, zeros32, idx3)

    return _pool(acc, deg, b5r, batch2d, Wfc, bfcr)


# speedup vs baseline: 1.1812x; 1.0374x over previous
import functools

import jax
import jax.numpy as jnp
from jax import lax
from jax.experimental import pallas as pl
from jax.experimental.pallas import tpu as pltpu
from jax.experimental.pallas import tpu_sc as plsc

N = 10000
E = 320000
NG = 64
NPAD = 10112
SLICE = NPAD // 16
E_PAD = 327680
EROWS = E_PAD // 128
ROWS_PER_TILE = EROWS // 32
CH = 8
CHB = 8
RMM = 1264
R = 2000
F32 = jnp.float32

_mesh = plsc.VectorSubcoreMesh(core_axis_name="c", subcore_axis_name="s")


def _deg_body(dst_hbm, out_hbm, dst_v, ones_v, zero_v, drain_v, acc_sh, ssem):
    c = lax.axis_index("c")
    s = lax.axis_index("s")
    for j in range(640 // 16):
        zero_v[pl.ds(j * 16, 16)] = jnp.zeros((16,), F32)
    for j in range(128 // 16):
        ones_v[pl.ds(j * 16, 16)] = jnp.full((16,), 1.0, F32)
    pltpu.sync_copy(zero_v, acc_sh.at[pl.ds(s * 640, 640)])
    plsc.subcore_barrier()

    row0 = c * (EROWS // 2) + s * ROWS_PER_TILE

    def body(i, carry):
        pltpu.sync_copy(dst_hbm.at[pl.ds(row0 + i * CH, CH)], dst_v)
        for j in range(CH):
            pltpu.async_copy(ones_v, acc_sh.at[dst_v.at[j]], ssem, add=True)
        return carry

    lax.fori_loop(0, ROWS_PER_TILE // CH, body, 0)
    pltpu.make_async_copy(out_hbm.at[0, pl.ds(0, ROWS_PER_TILE * 128)],
                          drain_v, ssem).wait()
    plsc.subcore_barrier()
    pltpu.sync_copy(acc_sh.at[pl.ds(s * 640, 640)],
                    out_hbm.at[c, pl.ds(s * 640, 640)])


_deg_kernel = functools.partial(
    pl.kernel,
    out_type=jax.ShapeDtypeStruct((2, 10240), F32),
    mesh=_mesh,
    scratch_types=[
        pltpu.VMEM((CH, 128), jnp.int32),
        pltpu.VMEM((128,), F32),
        pltpu.VMEM((640,), F32),
        pltpu.VMEM((ROWS_PER_TILE * 128,), F32),
        pltpu.VMEM_SHARED((10240,), F32),
        pltpu.SemaphoreType.DMA,
    ],
)(_deg_body)


def _agg2_body(g_hbm, zeros_hbm, idx_hbm, out_hbm,
               idx_v, r0, r1, acc_sh, g0, g1, s0, s1):
    c = lax.axis_index("c")
    s = lax.axis_index("s")
    @pl.when(c == 0)
    def _():
        pltpu.sync_copy(g_hbm.at[pl.ds(s * SLICE, SLICE)],
                        acc_sh.at[pl.ds(s * SLICE, SLICE)])

    @pl.when(c == 1)
    def _():
        pltpu.sync_copy(zeros_hbm.at[pl.ds(s * SLICE, SLICE)],
                        acc_sh.at[pl.ds(s * SLICE, SLICE)])

    plsc.subcore_barrier()

    rows = (r0, r1)
    gsem = (g0, g1)
    ssem = (s0, s1)
    row_base = c * (EROWS // 2) + s * ROWS_PER_TILE
    half = ROWS_PER_TILE // 2

    def _wait_g(b):
        pltpu.make_async_copy(g_hbm.at[pl.ds(0, 128)], rows[b], gsem[b]).wait()

    def _wait_s(b):
        pltpu.make_async_copy(g_hbm.at[pl.ds(0, 128)], rows[b], ssem[b]).wait()

    def _run_half(hbase, first_half):
        if not first_half:
            _wait_s(0)
            _wait_s(1)
        pltpu.sync_copy(idx_hbm.at[pl.ds(row_base + hbase, half)], idx_v)
        pltpu.async_copy(g_hbm.at[idx_v.at[0, 0]], rows[0], gsem[0])

        def body(k, carry):
            @pl.when(k > 0)
            def _():
                _wait_s(1)
            pltpu.async_copy(g_hbm.at[idx_v.at[2 * k + 1, 0]],
                             rows[1], gsem[1])
            _wait_g(0)
            pltpu.async_copy(rows[0], acc_sh.at[idx_v.at[2 * k, 1]],
                             ssem[0], add=True)

            @pl.when(k < half // 2 - 1)
            def _():
                _wait_s(0)
                pltpu.async_copy(g_hbm.at[idx_v.at[2 * k + 2, 0]],
                                 rows[0], gsem[0])
            _wait_g(1)
            pltpu.async_copy(rows[1], acc_sh.at[idx_v.at[2 * k + 1, 1]],
                             ssem[1], add=True)
            return carry

        lax.fori_loop(0, half // 2, body, 0)

    _run_half(0, True)
    _run_half(half, False)
    _wait_s(0)
    _wait_s(1)
    plsc.subcore_barrier()
    pltpu.sync_copy(acc_sh.at[pl.ds(s * SLICE, SLICE)],
                    out_hbm.at[c, pl.ds(s * SLICE, SLICE)])


_agg128 = functools.partial(
    pl.kernel,
    out_type=jax.ShapeDtypeStruct((2, NPAD, 128), F32),
    mesh=_mesh,
    compiler_params=pltpu.CompilerParams(use_tc_tiling_on_sc=False),
    scratch_types=(
        [pltpu.VMEM((ROWS_PER_TILE // 2, 2, 128), jnp.int32)]
        + [pltpu.VMEM((128, 128), F32) for _ in range(2)]
        + [pltpu.VMEM_SHARED((NPAD, 128), F32)]
        + [pltpu.SemaphoreType.DMA] * 4
    ),
)(_agg2_body)


def _aggn_body(fw, g_hbm, zeros_hbm, idx_hbm, out_hbm,
               idx_v, r0, r1, r2, r3, acc_sh,
               g0, g1, g2, g3, s0, s1, s2, s3):
    c = lax.axis_index("c")
    s = lax.axis_index("s")
    @pl.when(c == 0)
    def _():
        pltpu.sync_copy(g_hbm.at[pl.ds(s * SLICE, SLICE)],
                        acc_sh.at[pl.ds(s * SLICE, SLICE)])

    @pl.when(c == 1)
    def _():
        pltpu.sync_copy(zeros_hbm.at[pl.ds(s * SLICE, SLICE)],
                        acc_sh.at[pl.ds(s * SLICE, SLICE)])

    plsc.subcore_barrier()

    rows = (r0, r1, r2, r3)
    gsem = (g0, g1, g2, g3)
    ssem = (s0, s1, s2, s3)
    row_base = c * (EROWS // 2) + s * ROWS_PER_TILE
    pltpu.sync_copy(idx_hbm.at[pl.ds(row_base, ROWS_PER_TILE)], idx_v)

    def _wait_g(b):
        pltpu.make_async_copy(g_hbm.at[pl.ds(0, 128)], rows[b], gsem[b]).wait()

    def _wait_s(b):
        pltpu.make_async_copy(g_hbm.at[pl.ds(0, 128)], rows[b], ssem[b]).wait()

    for j in range(3):
        pltpu.async_copy(g_hbm.at[idx_v.at[j, 0]], rows[j], gsem[j])

    def quad(k, carry):
        for u4 in range(4):
            j = 4 * k + u4
            b = u4
            b2 = (u4 + 3) % 4
            _wait_g(b)
            pltpu.async_copy(rows[b], acc_sh.at[idx_v.at[j, 1]],
                             ssem[b], add=True)
            j2 = j + 3

            @pl.when(jnp.logical_and(j >= 1, j2 < ROWS_PER_TILE))
            def _():
                _wait_s(b2)

            @pl.when(j2 < ROWS_PER_TILE)
            def _():
                pltpu.async_copy(g_hbm.at[idx_v.at[j2, 0]],
                                 rows[b2], gsem[b2])
        return carry

    lax.fori_loop(0, ROWS_PER_TILE // 4, quad, 0)
    for b in range(4):
        _wait_s(b)
    plsc.subcore_barrier()
    pltpu.sync_copy(acc_sh.at[pl.ds(s * SLICE, SLICE)],
                    out_hbm.at[c, pl.ds(s * SLICE, SLICE)])


def _make_aggn(fw):
    return functools.partial(
        pl.kernel,
        out_type=jax.ShapeDtypeStruct((2, NPAD, fw), F32),
        mesh=_mesh,
        compiler_params=pltpu.CompilerParams(use_tc_tiling_on_sc=False),
        scratch_types=(
            [pltpu.VMEM((ROWS_PER_TILE, 2, 128), jnp.int32)]
            + [pltpu.VMEM((128, fw), F32) for _ in range(4)]
            + [pltpu.VMEM_SHARED((NPAD, fw), F32)]
            + [pltpu.SemaphoreType.DMA] * 8
        ),
    )(functools.partial(_aggn_body, fw))


_agg64 = _make_aggn(64)
_agg32 = _make_aggn(32)


def _mm_first_body(x_ref, w_ref, deg_ref, o_ref):
    d = deg_ref[:, 0] + deg_ref[:, 1] + 1.0
    dinv = lax.rsqrt(d)
    g = jnp.dot(x_ref[...], w_ref[...], preferred_element_type=F32)
    o_ref[...] = g * dinv[:, None]


def _mm_mid_body(acc_ref, deg_ref, b_ref, w_ref, o_ref):
    d = deg_ref[:, 0] + deg_ref[:, 1] + 1.0
    dinv = lax.rsqrt(d)
    h = acc_ref[0] + acc_ref[1]
    a = jnp.maximum(dinv[:, None] * h + b_ref[...], 0.0)
    g = jnp.dot(a, w_ref[...], preferred_element_type=F32)
    o_ref[...] = g * dinv[:, None]


def _pool_body(acc_ref, deg_ref, b_ref, batch_ref, wfc_ref, bfc_ref,
               o_ref, pooled_scr):
    i = pl.program_id(0)

    @pl.when(i == 0)
    def _():
        pooled_scr[...] = jnp.zeros((NG, NG), F32)

    d = deg_ref[:, 0] + deg_ref[:, 1] + 1.0
    dinv = lax.rsqrt(d)
    h = acc_ref[0] + acc_ref[1]
    a = jnp.maximum(dinv[:, None] * h + b_ref[...], 0.0)
    ax = jnp.concatenate([a, jnp.ones((R, 32), F32)], axis=-1)
    gid = batch_ref[...]
    oh = (gid == lax.broadcasted_iota(jnp.int32, (R, NG), 1)).astype(F32)
    pooled_scr[...] += lax.dot_general(
        oh, ax, (((0,), (0,)), ((), ())), preferred_element_type=F32)

    @pl.when(i == pl.num_programs(0) - 1)
    def _():
        pooled = pooled_scr[...]
        counts = jnp.maximum(pooled[:, 32:33], 1.0)
        mean = pooled[:, :32] / counts
        o_ref[...] = jnp.dot(mean, wfc_ref[...],
                             preferred_element_type=F32) + bfc_ref[...]


def _mm_first(x, w, deg):
    return pl.pallas_call(
        _mm_first_body,
        grid=(NPAD // RMM,),
        in_specs=[
            pl.BlockSpec((RMM, x.shape[1]), lambda i: (i, 0)),
            pl.BlockSpec(w.shape, lambda i: (0, 0)),
            pl.BlockSpec((RMM, 2), lambda i: (i, 0)),
        ],
        out_specs=pl.BlockSpec((RMM, 128), lambda i: (i, 0)),
        out_shape=jax.ShapeDtypeStruct((NPAD, 128), F32),
    )(x, w, deg)


def _mm_mid(acc, deg, b, w):
    fin = acc.shape[2]
    fout = w.shape[1]
    return pl.pallas_call(
        _mm_mid_body,
        grid=(NPAD // RMM,),
        in_specs=[
            pl.BlockSpec((2, RMM, fin), lambda i: (0, i, 0)),
            pl.BlockSpec((RMM, 2), lambda i: (i, 0)),
            pl.BlockSpec(b.shape, lambda i: (0, 0)),
            pl.BlockSpec(w.shape, lambda i: (0, 0)),
        ],
        out_specs=pl.BlockSpec((RMM, fout), lambda i: (i, 0)),
        out_shape=jax.ShapeDtypeStruct((NPAD, fout), F32),
    )(acc, deg, b, w)


def _pool(acc, deg, b, batch2d, wfc, bfc):
    return pl.pallas_call(
        _pool_body,
        grid=(N // R,),
        in_specs=[
            pl.BlockSpec((2, R, 32), lambda i: (0, i, 0)),
            pl.BlockSpec((R, 2), lambda i: (i, 0)),
            pl.BlockSpec(b.shape, lambda i: (0, 0)),
            pl.BlockSpec((R, 1), lambda i: (i, 0)),
            pl.BlockSpec(wfc.shape, lambda i: (0, 0)),
            pl.BlockSpec(bfc.shape, lambda i: (0, 0)),
        ],
        out_specs=pl.BlockSpec((NG, 10), lambda i: (0, 0)),
        out_shape=jax.ShapeDtypeStruct((NG, 10), F32),
        scratch_shapes=[pltpu.VMEM((NG, NG), F32)],
    )(acc, deg, b, batch2d, wfc, bfc)


def kernel(x, edge_index, batch, W1, b1, W2, b2, W3, b3, W4, b4, W5, b5,
           Wfc, bfc):
    src = edge_index[0].astype(jnp.int32)
    dst = edge_index[1].astype(jnp.int32)
    npad = E_PAD - E
    pad_src = (jnp.arange(npad, dtype=jnp.int32) * 13) % N
    pad_dst = N + (jnp.arange(npad, dtype=jnp.int32) % (NPAD - N))
    src2 = jnp.concatenate([src, pad_src]).reshape(EROWS, 128)
    dst2 = jnp.concatenate([dst, pad_dst]).reshape(EROWS, 128)
    idx3 = jnp.stack([src2, dst2], axis=1)

    batch2d = batch.astype(jnp.int32).reshape(N, 1)
    zeros = jnp.zeros((NPAD, 128), F32)
    zeros64 = jnp.zeros((NPAD, 64), F32)
    zeros32 = jnp.zeros((NPAD, 32), F32)
    b1r = b1.reshape(1, -1)
    b2r = b2.reshape(1, -1)
    b3r = b3.reshape(1, -1)
    b4r = b4.reshape(1, -1)
    b5r = b5.reshape(1, -1)
    bfcr = bfc.reshape(1, -1)

    deg = _deg_kernel(dst2).T

    g = _mm_first(x, W1, deg)
    acc = _agg128(g, zeros, idx3)
    g = _mm_mid(acc, deg, b1r, W2)
    acc = _agg128(g, zeros, idx3)
    g = _mm_mid(acc, deg, b2r, W3)
    acc = _agg128(g, zeros, idx3)
    g = _mm_mid(acc, deg, b3r, W4)
    acc = _agg64(g, zeros64, idx3)
    g = _mm_mid(acc, deg, b4r, W5)
    acc = _agg32(g, zeros32, idx3)

    return _pool(acc, deg, b5r, batch2d, Wfc, bfcr)
